# baseline (speedup 1.0000x reference)
import functools

import jax
import jax.numpy as jnp
from jax import lax
from jax.experimental import pallas as pl
from jax.experimental.pallas import tpu as pltpu
from jax.experimental.pallas import tpu_sc as plsc

N = 10000
F_IN = 128
D = 32
G = 64
C = 10
E = 320000

NC = 2
NS = 16
NW = NC * NS
E_PAD = NW * 10240


def _ch(W):
    return 128
N_PAD = ((N + NS * 8) // (NS * 8)) * (NS * 8)
RPT = N_PAD // NS


def _sc_agg_body(W, ZR, CH, K, h_hbm, src_hbm, dst_hbm, out_hbm,
                 src_v, dst_v, rows_v, zero_v, acc_sh, gsem):
    c = lax.axis_index("c")
    s = lax.axis_index("s")
    wid = c * NS + s

    zero = jnp.zeros((16,), jnp.float32)

    def zfill(i, _):
        for q in range(W // 16):
            zero_v[i, pl.ds(q * 16, 16)] = zero
        return 0

    lax.fori_loop(0, ZR, zfill, 0)
    row0 = s * RPT
    for k in range(RPT // ZR):
        pltpu.sync_copy(zero_v, acc_sh.at[pl.ds(row0 + k * ZR, ZR)])
    plsc.subcore_barrier()

    base = wid * K

    def edge_chunk(j, _):
        pltpu.sync_copy(src_hbm.at[base + j], src_v)
        pltpu.sync_copy(dst_hbm.at[base + j], dst_v)
        pltpu.async_copy(h_hbm.at[src_v], rows_v, gsem).wait()
        pltpu.sync_copy(rows_v, acc_sh.at[dst_v], add=True)
        return 0

    lax.fori_loop(0, K, edge_chunk, 0)
    plsc.subcore_barrier()

    pltpu.sync_copy(acc_sh.at[pl.ds(row0, RPT)],
                    out_hbm.at[c, pl.ds(row0, RPT)])


@functools.cache
def _sc_agg(W):
    ZR = 632 if W <= 32 else 79
    CH = _ch(W)
    K = E_PAD // (NW * CH)
    return pl.kernel(
        functools.partial(_sc_agg_body, W, ZR, CH, K),
        out_type=jax.ShapeDtypeStruct((NC, N_PAD, W), jnp.float32),
        mesh=plsc.VectorSubcoreMesh(core_axis_name="c", subcore_axis_name="s",
                                    num_cores=NC, num_subcores=NS),
        compiler_params=pltpu.CompilerParams(use_tc_tiling_on_sc=False),
        scratch_types=[
            pltpu.VMEM((CH,), jnp.int32),
            pltpu.VMEM((CH,), jnp.int32),
            pltpu.VMEM((CH, W), jnp.float32),
            pltpu.VMEM((ZR, W), jnp.float32),
            pltpu.VMEM_SHARED((N_PAD, W), jnp.float32),
            pltpu.SemaphoreType.DMA,
        ],
    )


def _layer_head(h, agg_ref, wa, ba, wb, bb, g, be):
    W = h.shape[1]
    z = h + agg_ref[0:N, :] + agg_ref[N_PAD:N_PAD + N, :]
    z = jnp.maximum(jnp.dot(z, wa[...], preferred_element_type=jnp.float32)
                    + ba[...], 0.0)
    z = jnp.dot(z, wb[...], preferred_element_type=jnp.float32) + bb[...]
    h2 = jnp.maximum(z, 0.0)
    m = jnp.mean(h2, axis=0, keepdims=True)
    v = jnp.mean((h2 - m) * (h2 - m), axis=0, keepdims=True)
    return (h2 - m) / jnp.sqrt(v + 1e-5) * g[...] + be[...]


def _mid_body(h_ref, agg_ref, wa, ba, wb, bb, g, be, o_ref):
    o_ref[...] = _layer_head(h_ref[...], agg_ref, wa, ba, wb, bb, g, be)


def _final_body(h_ref, agg_ref, wa, ba, wb, bb, g, be, batch_ref,
                fc1w, fc1b, fc2w, fc2b, o_ref):
    hn = _layer_head(h_ref[...], agg_ref, wa, ba, wb, bb, g, be)
    seg = lax.broadcasted_iota(jnp.int32, (G, N), 0)
    onehot = (seg == batch_ref[...]).astype(jnp.float32)
    pooled = jnp.dot(onehot, hn, preferred_element_type=jnp.float32,
                     precision=lax.Precision.HIGHEST)
    p = jnp.maximum(
        jnp.dot(pooled, fc1w[...], preferred_element_type=jnp.float32)
        + fc1b[...], 0.0)
    o = jnp.dot(p, fc2w[...], preferred_element_type=jnp.float32) + fc2b[...]
    o = o - jnp.max(o, axis=-1, keepdims=True)
    o_ref[...] = o - jnp.log(jnp.sum(jnp.exp(o), axis=-1, keepdims=True))


def _tc_call(body, out_shape):
    return pl.pallas_call(body, out_shape=jax.ShapeDtypeStruct(out_shape,
                                                               jnp.float32))


def kernel(x, params, edge_index, batch):
    pad = E_PAD - E
    perm = jnp.argsort(edge_index[1], stable=True)
    src_p = jnp.concatenate([edge_index[0][perm],
                             jnp.zeros((pad,), jnp.int32)])
    dst_p = jnp.concatenate([edge_index[1][perm],
                             jnp.full((pad,), N, jnp.int32)])

    def row(v):
        return v.reshape(1, -1)

    h = x
    for i in range(5):
        W = h.shape[1]
        ch = _ch(W)
        agg = _sc_agg(W)(h, src_p.reshape(-1, ch),
                         dst_p.reshape(-1, ch)).reshape(NC * N_PAD, W)
        wa, ba = params[f"w{i}a"], params[f"b{i}a"]
        wb, bb = params[f"w{i}b"], params[f"b{i}b"]
        g, be = params[f"g{i}"], params[f"be{i}"]
        if i < 4:
            h = _tc_call(_mid_body, (N, D))(
                h, agg, wa, row(ba), wb, row(bb), row(g), row(be))
        else:
            out = _tc_call(_final_body, (G, C))(
                h, agg, wa, row(ba), wb, row(bb), row(g), row(be),
                row(batch), params["fc1_w"], row(params["fc1_b"]),
                params["fc2_w"], row(params["fc2_b"]))
    return out


# speedup vs baseline: 1.2569x; 1.2569x over previous
import functools

import jax
import jax.numpy as jnp
from jax import lax
from jax.experimental import pallas as pl
from jax.experimental.pallas import tpu as pltpu
from jax.experimental.pallas import tpu_sc as plsc

N = 10000
F_IN = 128
D = 32
G = 64
C = 10
E = 320000

NC = 2
NS = 16
NW = NC * NS
CH = 128
K = (E + NW * CH - 1) // (NW * CH)
E_PAD = NW * K * CH
N_PAD = ((N + NS * 8) // (NS * 8)) * (NS * 8)
RPT = N_PAD // NS


def _sc_agg_body(W, h_hbm, src_hbm, dst_hbm, out_hbm,
                 src_v, dst_v, rows_v, zero_v, acc_sh, sem):
    c = lax.axis_index("c")
    s = lax.axis_index("s")
    wid = c * NS + s

    zero = jnp.zeros((16,), jnp.float32)
    for i in range(8):
        for q in range(W // 16):
            zero_v[i, pl.ds(q * 16, 16)] = zero
    row0 = s * RPT

    def zero_chunk(k, _):
        pltpu.sync_copy(zero_v, acc_sh.at[pl.ds(row0 + k * 8, 8)])
        return 0

    lax.fori_loop(0, RPT // 8, zero_chunk, 0)
    plsc.subcore_barrier()

    base = wid * (K * CH)

    def edge_chunk(j, _):
        off = base + j * CH
        pltpu.sync_copy(src_hbm.at[pl.ds(off, CH)], src_v)
        pltpu.sync_copy(dst_hbm.at[pl.ds(off, CH)], dst_v)
        pltpu.async_copy(h_hbm.at[src_v], rows_v, sem).wait()
        pltpu.sync_copy(rows_v, acc_sh.at[dst_v], add=True)
        return 0

    lax.fori_loop(0, K, edge_chunk, 0)
    plsc.subcore_barrier()

    pltpu.sync_copy(acc_sh.at[pl.ds(row0, RPT)],
                    out_hbm.at[c, pl.ds(row0, RPT)])


@functools.cache
def _sc_agg(W):
    return pl.kernel(
        functools.partial(_sc_agg_body, W),
        out_type=jax.ShapeDtypeStruct((NC, N_PAD, W), jnp.float32),
        mesh=plsc.VectorSubcoreMesh(core_axis_name="c", subcore_axis_name="s",
                                    num_cores=NC, num_subcores=NS),
        compiler_params=pltpu.CompilerParams(use_tc_tiling_on_sc=False),
        scratch_types=[
            pltpu.VMEM((CH,), jnp.int32),
            pltpu.VMEM((CH,), jnp.int32),
            pltpu.VMEM((CH, W), jnp.float32),
            pltpu.VMEM((8, W), jnp.float32),
            pltpu.VMEM_SHARED((N_PAD, W), jnp.float32),
            pltpu.SemaphoreType.DMA,
        ],
    )


def _layer_head(h, agg_ref, wa, ba, wb, bb, g, be):
    W = h.shape[1]
    z = h + agg_ref[0:N, :] + agg_ref[N_PAD:N_PAD + N, :]
    z = jnp.maximum(jnp.dot(z, wa[...], preferred_element_type=jnp.float32)
                    + ba[...], 0.0)
    z = jnp.dot(z, wb[...], preferred_element_type=jnp.float32) + bb[...]
    h2 = jnp.maximum(z, 0.0)
    m = jnp.mean(h2, axis=0, keepdims=True)
    v = jnp.mean((h2 - m) * (h2 - m), axis=0, keepdims=True)
    return (h2 - m) / jnp.sqrt(v + 1e-5) * g[...] + be[...]


def _mid_body(h_ref, agg_ref, wa, ba, wb, bb, g, be, o_ref):
    o_ref[...] = _layer_head(h_ref[...], agg_ref, wa, ba, wb, bb, g, be)


def _final_body(h_ref, agg_ref, wa, ba, wb, bb, g, be, batch_ref,
                fc1w, fc1b, fc2w, fc2b, o_ref):
    hn = _layer_head(h_ref[...], agg_ref, wa, ba, wb, bb, g, be)
    seg = lax.broadcasted_iota(jnp.int32, (G, N), 0)
    onehot = (seg == batch_ref[...]).astype(jnp.float32)
    pooled = jnp.dot(onehot, hn, preferred_element_type=jnp.float32,
                     precision=lax.Precision.HIGHEST)
    p = jnp.maximum(
        jnp.dot(pooled, fc1w[...], preferred_element_type=jnp.float32)
        + fc1b[...], 0.0)
    o = jnp.dot(p, fc2w[...], preferred_element_type=jnp.float32) + fc2b[...]
    o = o - jnp.max(o, axis=-1, keepdims=True)
    o_ref[...] = o - jnp.log(jnp.sum(jnp.exp(o), axis=-1, keepdims=True))


def _tc_call(body, out_shape):
    return pl.pallas_call(body, out_shape=jax.ShapeDtypeStruct(out_shape,
                                                               jnp.float32))


def kernel(x, params, edge_index, batch):
    pad = E_PAD - E
    perm = jnp.argsort(edge_index[1], stable=True)
    src_p = jnp.concatenate([edge_index[0][perm],
                             jnp.zeros((pad,), jnp.int32)])
    dst_p = jnp.concatenate([edge_index[1][perm],
                             jnp.full((pad,), N, jnp.int32)])

    def row(v):
        return v.reshape(1, -1)

    h = x
    for i in range(5):
        W = h.shape[1]
        agg = _sc_agg(W)(h, src_p, dst_p).reshape(NC * N_PAD, W)
        wa, ba = params[f"w{i}a"], params[f"b{i}a"]
        wb, bb = params[f"w{i}b"], params[f"b{i}b"]
        g, be = params[f"g{i}"], params[f"be{i}"]
        if i < 4:
            h = _tc_call(_mid_body, (N, D))(
                h, agg, wa, row(ba), wb, row(bb), row(g), row(be))
        else:
            out = _tc_call(_final_body, (G, C))(
                h, agg, wa, row(ba), wb, row(bb), row(g), row(be),
                row(batch), params["fc1_w"], row(params["fc1_b"]),
                params["fc2_w"], row(params["fc2_b"]))
    return out


# speedup vs baseline: 1.3938x; 1.1090x over previous
import functools

import jax
import jax.numpy as jnp
from jax import lax
from jax.experimental import pallas as pl
from jax.experimental.pallas import tpu as pltpu
from jax.experimental.pallas import tpu_sc as plsc

N = 10000
F_IN = 128
D = 32
G = 64
C = 10
E = 320000

NC = 2
NS = 16
NW = NC * NS
CH = 128
K = (E + NW * CH - 1) // (NW * CH)
E_PAD = NW * K * CH
N_PAD = ((N + NS * 8) // (NS * 8)) * (NS * 8)
RPT = N_PAD // NS


def _sc_agg_body(W, h_hbm, src_hbm, dst_hbm, out_hbm,
                 src_v, dst_v, rows_v, zero_v, acc_sh, sem):
    c = lax.axis_index("c")
    s = lax.axis_index("s")
    wid = c * NS + s

    zero = jnp.zeros((16,), jnp.float32)
    for i in range(8):
        for q in range(W // 16):
            zero_v[i, pl.ds(q * 16, 16)] = zero
    row0 = s * RPT

    def zero_chunk(k, _):
        pltpu.sync_copy(zero_v, acc_sh.at[pl.ds(row0 + k * 8, 8)])
        return 0

    lax.fori_loop(0, RPT // 8, zero_chunk, 0)
    plsc.subcore_barrier()

    base = wid * (K * CH)

    def edge_chunk(j, _):
        off = base + j * CH
        ca = pltpu.async_copy(src_hbm.at[pl.ds(off, CH)], src_v, sem)
        cb = pltpu.async_copy(dst_hbm.at[pl.ds(off, CH)], dst_v, sem)
        ca.wait()
        cb.wait()
        pltpu.async_copy(h_hbm.at[src_v], rows_v, sem).wait()
        pltpu.sync_copy(rows_v, acc_sh.at[dst_v], add=True)
        return 0

    lax.fori_loop(0, K, edge_chunk, 0)
    plsc.subcore_barrier()

    pltpu.sync_copy(acc_sh.at[pl.ds(row0, RPT)],
                    out_hbm.at[c, pl.ds(row0, RPT)])


@functools.cache
def _sc_agg(W):
    return pl.kernel(
        functools.partial(_sc_agg_body, W),
        out_type=jax.ShapeDtypeStruct((NC, N_PAD, W), jnp.float32),
        mesh=plsc.VectorSubcoreMesh(core_axis_name="c", subcore_axis_name="s",
                                    num_cores=NC, num_subcores=NS),
        compiler_params=pltpu.CompilerParams(use_tc_tiling_on_sc=False),
        scratch_types=[
            pltpu.VMEM((CH,), jnp.int32),
            pltpu.VMEM((CH,), jnp.int32),
            pltpu.VMEM((CH, W), jnp.float32),
            pltpu.VMEM((8, W), jnp.float32),
            pltpu.VMEM_SHARED((N_PAD, W), jnp.float32),
            pltpu.SemaphoreType.DMA,
        ],
    )


def _layer_head(h, agg_ref, wa, ba, wb, bb, g, be):
    W = h.shape[1]
    z = h + agg_ref[0:N, :] + agg_ref[N_PAD:N_PAD + N, :]
    z = jnp.maximum(jnp.dot(z, wa[...], preferred_element_type=jnp.float32)
                    + ba[...], 0.0)
    z = jnp.dot(z, wb[...], preferred_element_type=jnp.float32) + bb[...]
    h2 = jnp.maximum(z, 0.0)
    m = jnp.mean(h2, axis=0, keepdims=True)
    v = jnp.mean((h2 - m) * (h2 - m), axis=0, keepdims=True)
    return (h2 - m) / jnp.sqrt(v + 1e-5) * g[...] + be[...]


def _mid_body(h_ref, agg_ref, wa, ba, wb, bb, g, be, o_ref):
    o_ref[...] = _layer_head(h_ref[...], agg_ref, wa, ba, wb, bb, g, be)


def _final_body(h_ref, agg_ref, wa, ba, wb, bb, g, be, batch_ref,
                fc1w, fc1b, fc2w, fc2b, o_ref):
    hn = _layer_head(h_ref[...], agg_ref, wa, ba, wb, bb, g, be)
    seg = lax.broadcasted_iota(jnp.int32, (G, N), 0)
    onehot = (seg == batch_ref[...]).astype(jnp.float32)
    pooled = jnp.dot(onehot, hn, preferred_element_type=jnp.float32,
                     precision=lax.Precision.HIGHEST)
    p = jnp.maximum(
        jnp.dot(pooled, fc1w[...], preferred_element_type=jnp.float32)
        + fc1b[...], 0.0)
    o = jnp.dot(p, fc2w[...], preferred_element_type=jnp.float32) + fc2b[...]
    o = o - jnp.max(o, axis=-1, keepdims=True)
    o_ref[...] = o - jnp.log(jnp.sum(jnp.exp(o), axis=-1, keepdims=True))


def _tc_call(body, out_shape):
    return pl.pallas_call(body, out_shape=jax.ShapeDtypeStruct(out_shape,
                                                               jnp.float32))


def kernel(x, params, edge_index, batch):
    pad = E_PAD - E
    perm = jnp.argsort(edge_index[1], stable=True)
    src_p = jnp.concatenate([edge_index[0][perm],
                             jnp.zeros((pad,), jnp.int32)])
    dst_p = jnp.concatenate([edge_index[1][perm],
                             jnp.full((pad,), N, jnp.int32)])

    def row(v):
        return v.reshape(1, -1)

    h = x
    for i in range(5):
        W = h.shape[1]
        agg = _sc_agg(W)(h, src_p, dst_p).reshape(NC * N_PAD, W)
        wa, ba = params[f"w{i}a"], params[f"b{i}a"]
        wb, bb = params[f"w{i}b"], params[f"b{i}b"]
        g, be = params[f"g{i}"], params[f"be{i}"]
        if i < 4:
            h = _tc_call(_mid_body, (N, D))(
                h, agg, wa, row(ba), wb, row(bb), row(g), row(be))
        else:
            out = _tc_call(_final_body, (G, C))(
                h, agg, wa, row(ba), wb, row(bb), row(g), row(be),
                row(batch), params["fc1_w"], row(params["fc1_b"]),
                params["fc2_w"], row(params["fc2_b"]))
    return out
